# baseline (speedup 1.0000x reference)
import functools

import jax
import jax.numpy as jnp
from jax import lax
from jax.experimental import pallas as pl
from jax.experimental.pallas import tpu as pltpu
from jax.experimental.pallas import tpu_sc as plsc

BATCH = 4096
SEQ = 200
HIDDEN = 64
VOCAB = 1000000
PADW = 128

NC = 2
NS = 16
NW = NC * NS

TOTAL = BATCH * SEQ
PER_W = TOTAL // NW
GRP = 128
K = 4
CHUNK = K * GRP
NOUT = PER_W // (2 * CHUNK)


@functools.cache
def _build_gather_kernel():
    mesh = plsc.VectorSubcoreMesh(core_axis_name="c", subcore_axis_name="s")
    return functools.partial(
        pl.kernel,
        mesh=mesh,
        out_type=jax.ShapeDtypeStruct((TOTAL, PADW), jnp.float32),
        scratch_types=[
            pltpu.VMEM((2 * K, GRP), jnp.int32),
            pltpu.VMEM((CHUNK, PADW), jnp.float32),
            pltpu.SemaphoreType.DMA,
        ],
        compiler_params=pltpu.CompilerParams(use_tc_tiling_on_sc=True),
    )(_gather_body)


def _gather_body(idx_hbm, table_hbm, out_hbm, idx_v, rows_v, sem):
    wid = lax.axis_index("s") * NC + lax.axis_index("c")
    base = wid * PER_W
    base_g = wid * (PER_W // GRP)

    def body(j, carry):
        pltpu.sync_copy(idx_hbm.at[pl.ds(base_g + j * 2 * K, 2 * K)], idx_v)
        for half in range(2):
            off = base + (2 * j + half) * CHUNK
            copies = []
            for g in range(K):
                copies.append(
                    pltpu.async_copy(
                        table_hbm.at[idx_v.at[half * K + g]],
                        rows_v.at[pl.ds(g * GRP, GRP)],
                        sem,
                    )
                )
            for c in copies:
                c.wait()
            pltpu.sync_copy(rows_v, out_hbm.at[pl.ds(off, CHUNK)])
        return carry

    lax.fori_loop(0, NOUT, body, 0)


def kernel(input_ids, W):
    idx = input_ids.reshape(TOTAL // GRP, GRP).astype(jnp.int32)
    Wp = jnp.pad(W, ((0, 0), (0, PADW - HIDDEN)))
    out = _build_gather_kernel()(idx, Wp)
    return out[:, :HIDDEN].reshape(BATCH, SEQ, HIDDEN)


# speedup vs baseline: 1.0395x; 1.0395x over previous
import functools

import jax
import jax.numpy as jnp
from jax import lax
from jax.experimental import pallas as pl
from jax.experimental.pallas import tpu as pltpu
from jax.experimental.pallas import tpu_sc as plsc

BATCH = 4096
SEQ = 200
HIDDEN = 64
VOCAB = 1000000
PADW = 128

NC = 2
NS = 16
NW = NC * NS

TOTAL = BATCH * SEQ
PER_W = TOTAL // NW
GRP = 128
CHUNK = 2 * GRP
NCHUNK = PER_W // CHUNK
IDXROWS = PER_W // GRP


@functools.cache
def _build_gather_kernel():
    mesh = plsc.VectorSubcoreMesh(core_axis_name="c", subcore_axis_name="s")
    return functools.partial(
        pl.kernel,
        mesh=mesh,
        out_type=jax.ShapeDtypeStruct((TOTAL, PADW), jnp.float32),
        scratch_types=[
            pltpu.VMEM((IDXROWS, GRP), jnp.int32),
            pltpu.VMEM((2, CHUNK, PADW), jnp.float32),
            pltpu.SemaphoreType.DMA,
            pltpu.SemaphoreType.DMA,
            pltpu.SemaphoreType.DMA,
            pltpu.SemaphoreType.DMA,
        ],
        compiler_params=pltpu.CompilerParams(use_tc_tiling_on_sc=True),
    )(_gather_body)


def _gather_body(idx_hbm, table_hbm, out_hbm, idx_v, rows_v,
                 sem_g0, sem_g1, sem_o0, sem_o1):
    wid = lax.axis_index("s") * NC + lax.axis_index("c")
    base = wid * PER_W
    gsems = (sem_g0, sem_g1)
    osems = (sem_o0, sem_o1)

    pltpu.sync_copy(idx_hbm.at[pl.ds(wid * IDXROWS, IDXROWS)], idx_v)

    def fire(c, b):
        for g in range(2):
            pltpu.async_copy(
                table_hbm.at[idx_v.at[2 * c + g]],
                rows_v.at[b, pl.ds(g * GRP, GRP)],
                gsems[b],
            )

    def store(c, b):
        pltpu.async_copy(
            rows_v.at[b], out_hbm.at[pl.ds(base + c * CHUNK, CHUNK)], osems[b]
        )

    def wait_gathers(b):
        for g in range(2):
            pltpu.make_async_copy(
                table_hbm.at[pl.ds(0, GRP)],
                rows_v.at[b, pl.ds(g * GRP, GRP)],
                gsems[b],
            ).wait()

    def wait_store(b):
        pltpu.make_async_copy(
            rows_v.at[b], out_hbm.at[pl.ds(base, CHUNK)], osems[b]
        ).wait()

    fire(0, 0)
    fire(1, 1)
    wait_gathers(0)
    store(0, 0)
    wait_store(0)
    fire(2, 0)
    wait_gathers(1)
    store(1, 1)

    def body(t, carry):
        c0 = 2 * t
        wait_store(1)
        fire(c0 + 1, 1)
        wait_gathers(0)
        store(c0, 0)
        wait_store(0)
        fire(c0 + 2, 0)
        wait_gathers(1)
        store(c0 + 1, 1)
        return carry

    lax.fori_loop(1, NCHUNK // 2 - 1, body, 0)

    wait_store(1)
    fire(NCHUNK - 1, 1)
    wait_gathers(0)
    store(NCHUNK - 2, 0)
    wait_gathers(1)
    store(NCHUNK - 1, 1)
    wait_store(0)
    wait_store(1)


def kernel(input_ids, W):
    idx = input_ids.reshape(TOTAL // GRP, GRP).astype(jnp.int32)
    Wp = jnp.pad(W, ((0, 0), (0, PADW - HIDDEN)))
    out = _build_gather_kernel()(idx, Wp)
    return out[:, :HIDDEN].reshape(BATCH, SEQ, HIDDEN)
